# baseline (speedup 1.0000x reference)
import functools

import jax
import jax.numpy as jnp
from jax import lax
from jax.experimental import pallas as pl
from jax.experimental.pallas import tpu as pltpu
from jax.experimental.pallas import tpu_sc as plsc

N = 10000
E = 160000
D = 256
HALF = 128
NS = 16
NC = 2
EPW = E // NS
CH = 40
SUP = 5
SCH = 2000
SPC = SCH // CH
CCH = 200
NCHUNK = EPW // CH
RPT = 624
TAIL = N - NS * RPT
CP = 40
NW = NC * NS
ECW = E // NW
CROWS = 80
N_PAD = 10240
TILE = 1024
GRID = N_PAD // TILE


def _edge_body(ta, tb, srcg, dstg, dstr, s_out,
               idx_s2, idx_g2, idx_d2, ra0, rb0, ra1, rb1,
               semA0, semB0, semA1, semB1, acc):
    c = lax.axis_index("c")
    s = lax.axis_index("s")
    zero16 = jnp.zeros((16,), jnp.float32)

    def _zb(i, carry):
        for j in range(8):
            ra0[i, pl.ds(j * 16, 16)] = zero16
        return carry
    lax.fori_loop(0, CH, _zb, 0)

    def _za(k, carry):
        pltpu.sync_copy(ra0, acc.at[pl.ds(s * RPT + k * CH, CH)])
        return carry
    lax.fori_loop(0, RPT // CH, _za, 0)
    _ZT = RPT - CH * (RPT // CH)
    pltpu.sync_copy(ra0.at[pl.ds(0, _ZT)],
                    acc.at[pl.ds(s * RPT + CH * (RPT // CH), _ZT)])

    @pl.when(s == 0)
    def _():
        pltpu.sync_copy(ra0.at[pl.ds(0, TAIL)],
                        acc.at[pl.ds(NS * RPT, TAIL)])
    plsc.subcore_barrier()

    def _relu_add(ra, rb):
        def _cmp(i, cc):
            for j in range(8):
                sl = pl.ds(j * 16, 16)
                ra[i, sl] = jnp.maximum(ra[i, sl] + rb[i, sl], 0.0)
            return cc
        lax.fori_loop(0, CH, _cmp, 0)

    def _sup(sup, carry):
        pltpu.sync_copy(srcg.at[c, s, sup], idx_s2)
        pltpu.sync_copy(dstg.at[c, s, sup], idx_g2)
        pltpu.sync_copy(dstr.at[s, sup], idx_d2)
        pltpu.async_copy(ta.at[idx_g2.at[0]], ra0, semA0)
        pltpu.async_copy(tb.at[idx_s2.at[0]], rb0, semB0)

        def _pair(k, cc):
            g0 = 2 * k
            pltpu.async_copy(ta.at[idx_g2.at[g0 + 1]], ra1, semA1)
            pltpu.async_copy(tb.at[idx_s2.at[g0 + 1]], rb1, semB1)
            pltpu.make_async_copy(ta.at[idx_g2.at[g0]], ra0, semA0).wait()
            pltpu.make_async_copy(tb.at[idx_s2.at[g0]], rb0, semB0).wait()
            _relu_add(ra0, rb0)
            pltpu.sync_copy(ra0, acc.at[idx_d2.at[g0]], add=True)

            @pl.when(k < SPC // 2 - 1)
            def _():
                pltpu.async_copy(ta.at[idx_g2.at[g0 + 2]], ra0, semA0)
                pltpu.async_copy(tb.at[idx_s2.at[g0 + 2]], rb0, semB0)
            pltpu.make_async_copy(ta.at[idx_g2.at[g0 + 1]], ra1, semA1).wait()
            pltpu.make_async_copy(tb.at[idx_s2.at[g0 + 1]], rb1, semB1).wait()
            _relu_add(ra1, rb1)
            pltpu.sync_copy(ra1, acc.at[idx_d2.at[g0 + 1]], add=True)
            return cc
        lax.fori_loop(0, SPC // 2, _pair, 0)
        return carry

    lax.fori_loop(0, SUP, _sup, 0)
    plsc.subcore_barrier()

    def _piece(base, nr):
        pltpu.sync_copy(acc.at[pl.ds(base, nr)], rb0.at[pl.ds(0, nr)])
        pltpu.sync_copy(rb0.at[pl.ds(0, nr)], s_out.at[c, pl.ds(base, nr)])

    def _cok(k, carry):
        _piece(s * RPT + k * CP, CP)
        return carry
    lax.fori_loop(0, RPT // CP, _cok, 0)
    _piece(s * RPT + CP * (RPT // CP), RPT - CP * (RPT // CP))

    @pl.when(s == 0)
    def _():
        _piece(NS * RPT, TAIL)


def _make_edge_kernel():
    scratch = [
        pltpu.VMEM((SPC, CH), jnp.int32),
        pltpu.VMEM((SPC, CH), jnp.int32),
        pltpu.VMEM((SPC, CH), jnp.int32),
        pltpu.VMEM((CH, HALF), jnp.float32),
        pltpu.VMEM((CH, HALF), jnp.float32),
        pltpu.VMEM((CH, HALF), jnp.float32),
        pltpu.VMEM((CH, HALF), jnp.float32),
        pltpu.SemaphoreType.DMA,
        pltpu.SemaphoreType.DMA,
        pltpu.SemaphoreType.DMA,
        pltpu.SemaphoreType.DMA,
        pltpu.VMEM_SHARED((N, HALF), jnp.float32),
    ]
    mesh = plsc.VectorSubcoreMesh(core_axis_name="c", subcore_axis_name="s")
    return pl.kernel(
        _edge_body,
        out_type=jax.ShapeDtypeStruct((NC, N_PAD, HALF), jnp.float32),
        mesh=mesh,
        scratch_types=scratch,
        name="edge_scatter",
    )


def _count_body(dst, cnt_out, idx_d, ones, sem, cacc):
    c = lax.axis_index("c")
    s = lax.axis_index("s")
    zero16 = jnp.zeros((16,), jnp.float32)
    one16 = jnp.ones((16,), jnp.float32)

    def _zb(i, carry):
        for j in range(8):
            ones[i, pl.ds(j * 16, 16)] = zero16
        return carry
    lax.fori_loop(0, CCH, _zb, 0)

    def _za(k, carry):
        pltpu.sync_copy(ones, cacc.at[pl.ds(s * RPT + k * CCH, CCH)])
        return carry
    lax.fori_loop(0, RPT // CCH, _za, 0)
    _ZT = RPT - CCH * (RPT // CCH)
    pltpu.sync_copy(ones.at[pl.ds(0, _ZT)],
                    cacc.at[pl.ds(s * RPT + CCH * (RPT // CCH), _ZT)])

    @pl.when(s == 0)
    def _():
        pltpu.sync_copy(ones.at[pl.ds(0, TAIL)],
                        cacc.at[pl.ds(NS * RPT, TAIL)])

    def _ob(i, carry):
        for j in range(8):
            ones[i, pl.ds(j * 16, 16)] = one16
        return carry
    lax.fori_loop(0, CCH, _ob, 0)
    plsc.subcore_barrier()

    EHC = E // NC
    EPW2 = EHC // NS

    def _scat(base):
        pltpu.sync_copy(dst.at[pl.ds(base, CCH)], idx_d)
        pltpu.sync_copy(ones, cacc.at[idx_d], add=True)

    def _ck(k, carry):
        _scat(c * EHC + s * EPW2 + k * CCH)
        return carry
    lax.fori_loop(0, EPW2 // CCH, _ck, 0)
    _rem = EPW2 - CCH * (EPW2 // CCH)
    if _rem:
        def _zt(i, carry):
            for j in range(8):
                ones[i, pl.ds(j * 16, 16)] = zero16
            return carry
        lax.fori_loop(0, CCH - _rem, _zt, 0)
        _scat(c * EHC + (s + 1) * EPW2 - CH)
    plsc.subcore_barrier()

    def _piece(base, nr):
        pltpu.sync_copy(cacc.at[pl.ds(base, nr)], ones.at[pl.ds(0, nr)])
        pltpu.sync_copy(ones.at[pl.ds(0, nr)], cnt_out.at[c, pl.ds(base, nr)])

    def _cok(k, carry):
        _piece(s * RPT + k * CP, CP)
        return carry
    lax.fori_loop(0, RPT // CP, _cok, 0)
    _piece(s * RPT + CP * (RPT // CP), RPT - CP * (RPT // CP))

    @pl.when(s == 0)
    def _():
        _piece(NS * RPT, TAIL)


def _make_count_kernel():
    scratch = [
        pltpu.VMEM((CCH,), jnp.int32),
        pltpu.VMEM((CCH, HALF), jnp.float32),
        pltpu.SemaphoreType.DMA,
        pltpu.VMEM_SHARED((N, HALF), jnp.float32),
    ]
    mesh = plsc.VectorSubcoreMesh(core_axis_name="c", subcore_axis_name="s")
    return pl.kernel(
        _count_body,
        out_type=jax.ShapeDtypeStruct((NC, N_PAD, HALF), jnp.float32),
        mesh=mesh,
        scratch_types=scratch,
        name="edge_count",
    )


def _pre_body(h_ref, wa_ref, ba_ref, ta_ref, tb_ref):
    wa = wa_ref[...]
    wt = wa[:D] - wa[D:]
    wb = wa[D:]
    h = h_ref[...]
    a = jnp.dot(h, wt, preferred_element_type=jnp.float32,
                 precision=lax.Precision.HIGHEST) + ba_ref[...]
    b = jnp.dot(h, wb, preferred_element_type=jnp.float32,
                 precision=lax.Precision.HIGHEST)
    ta_ref[0] = a[:, :HALF]
    ta_ref[1] = a[:, HALF:]
    tb_ref[0] = b[:, :HALF]
    tb_ref[1] = b[:, HALF:]


_pre = pl.pallas_call(
    _pre_body,
    grid=(GRID,),
    in_specs=[
        pl.BlockSpec((TILE, D), lambda i: (i, 0)),
        pl.BlockSpec((2 * D, D), lambda i: (0, 0)),
        pl.BlockSpec((1, D), lambda i: (0, 0)),
    ],
    out_specs=[
        pl.BlockSpec((NC, TILE, HALF), lambda i: (0, i, 0)),
        pl.BlockSpec((NC, TILE, HALF), lambda i: (0, i, 0)),
    ],
    out_shape=[
        jax.ShapeDtypeStruct((NC, N_PAD, HALF), jnp.float32),
        jax.ShapeDtypeStruct((NC, N_PAD, HALF), jnp.float32),
    ],
)


def _post_body(s_ref, cnt_ref, wb_ref, bb_ref, o_ref):
    cv = cnt_ref[0][:, 0:1] + cnt_ref[1][:, 0:1]
    inv = 1.0 / jnp.maximum(cv, 1.0)
    wb = wb_ref[...]
    s0 = s_ref[0] * inv
    s1 = s_ref[1] * inv
    o = (jnp.dot(s0, wb[:HALF], preferred_element_type=jnp.float32,
                 precision=lax.Precision.HIGHEST)
         + jnp.dot(s1, wb[HALF:], preferred_element_type=jnp.float32,
                 precision=lax.Precision.HIGHEST))
    o_ref[...] = o + jnp.where(cv > 0.0, bb_ref[...], 0.0)


_post = pl.pallas_call(
    _post_body,
    grid=(GRID,),
    in_specs=[
        pl.BlockSpec((NC, TILE, HALF), lambda i: (0, i, 0)),
        pl.BlockSpec((NC, TILE, HALF), lambda i: (0, i, 0)),
        pl.BlockSpec((D, D), lambda i: (0, 0)),
        pl.BlockSpec((1, D), lambda i: (0, 0)),
    ],
    out_specs=pl.BlockSpec((TILE, D), lambda i: (i, 0)),
    out_shape=jax.ShapeDtypeStruct((N_PAD, D), jnp.float32),
)


def _heads_body(h_ref, wf1, bf1, wf2, bf2, ws1, bs1, ws2, bs2,
                wo1, bo1, wo2, bo2, wm1, bm1, wm2, bm2, o_ref):
    h = h_ref[...]

    def head(w1, b1, w2, b2):
        g = jnp.maximum(
            jnp.dot(h, w1[...], preferred_element_type=jnp.float32,
                 precision=lax.Precision.HIGHEST) + b1[...],
            0.0)
        return jnp.dot(g, w2[...], preferred_element_type=jnp.float32,
                 precision=lax.Precision.HIGHEST) + b2[...]

    f = head(wf1, bf1, wf2, bf2)
    sc = head(ws1, bs1, ws2, bs2)
    off = head(wo1, bo1, wo2, bo2)
    m = head(wm1, bm1, wm2, bm2)
    m = 1.0 / (1.0 + jnp.exp(-m))
    o_ref[...] = jnp.concatenate([f, sc, off, m], axis=-1)


def _w_spec(k):
    return pl.BlockSpec((D, k), lambda i: (0, 0))


def _b_spec(k):
    return pl.BlockSpec((1, k), lambda i: (0, 0))


_heads = pl.pallas_call(
    _heads_body,
    grid=(GRID,),
    in_specs=[
        pl.BlockSpec((TILE, D), lambda i: (i, 0)),
        _w_spec(D), _b_spec(D), _w_spec(32), _b_spec(32),
        _w_spec(D), _b_spec(D), _w_spec(6), _b_spec(6),
        _w_spec(D), _b_spec(D), _w_spec(30), _b_spec(30),
        _w_spec(D), _b_spec(D), _w_spec(11), _b_spec(11),
    ],
    out_specs=pl.BlockSpec((TILE, 79), lambda i: (i, 0)),
    out_shape=jax.ShapeDtypeStruct((N_PAD, 79), jnp.float32),
)


def kernel(x, edge_index, W1a, b1a, W1b, b1b, W2a, b2a, W2b, b2b,
           Wf1, bf1, Wf2, bf2, Ws1, bs1, Ws2, bs2,
           Wo1, bo1, Wo2, bo2, Wm1, bm1, Wm2, bm2):
    src = edge_index[0]
    dst = edge_index[1]
    r2 = lambda b: b.reshape(1, -1)
    xp = jnp.concatenate(
        [x, jnp.zeros((N_PAD - N, x.shape[1]), x.dtype)], axis=0)

    edge = _make_edge_kernel()
    count = _make_count_kernel()

    cntp = count(dst)
    src2 = jnp.concatenate([src, src + N_PAD]).reshape(NC, NS, SUP, SPC, CH)
    dst2 = jnp.concatenate([dst, dst + N_PAD]).reshape(NC, NS, SUP, SPC, CH)
    dst3 = dst.reshape(NS, SUP, SPC, CH)

    ta, tb = _pre(xp, W1a, r2(b1a))
    s_acc = edge(ta.reshape(NC * N_PAD, HALF),
                 tb.reshape(NC * N_PAD, HALF), src2, dst2, dst3)
    h = _post(s_acc, cntp, W1b, r2(b1b))

    ta, tb = _pre(h, W2a, r2(b2a))
    s_acc = edge(ta.reshape(NC * N_PAD, HALF),
                 tb.reshape(NC * N_PAD, HALF), src2, dst2, dst3)
    h = _post(s_acc, cntp, W2b, r2(b2b))

    return _heads(h, Wf1, r2(bf1), Wf2, r2(bf2), Ws1, r2(bs1), Ws2, r2(bs2),
                  Wo1, r2(bo1), Wo2, r2(bo2), Wm1, r2(bm1), Wm2, r2(bm2))[:N]


# speedup vs baseline: 1.1488x; 1.1488x over previous
import functools

import jax
import jax.numpy as jnp
from jax import lax
from jax.experimental import pallas as pl
from jax.experimental.pallas import tpu as pltpu
from jax.experimental.pallas import tpu_sc as plsc

N = 10000
E = 160000
D = 256
HALF = 128
NS = 16
NC = 2
EPW = E // NS
CH = 40
SUP = 5
SCH = 2000
SPC = SCH // CH
CCH = 200
NCHUNK = EPW // CH
RPT = 624
TAIL = N - NS * RPT
CP = 40
NW = NC * NS
ECW = E // NW
CROWS = 80
N_PAD = 10240
TILE = 1024
GRID = N_PAD // TILE


def _edge_body(ta, tb, srcg, dstg, dstr, s_out,
               idx_s2, idx_g2, idx_d2, ra0, rb0, ra1, rb1,
               semA0, semB0, semA1, semB1, acc):
    c = lax.axis_index("c")
    s = lax.axis_index("s")
    zero16 = jnp.zeros((16,), jnp.float32)

    def _zb(i, carry):
        for j in range(8):
            ra0[i, pl.ds(j * 16, 16)] = zero16
        return carry
    lax.fori_loop(0, CH, _zb, 0)

    def _za(k, carry):
        pltpu.sync_copy(ra0, acc.at[pl.ds(s * RPT + k * CH, CH)])
        return carry
    lax.fori_loop(0, RPT // CH, _za, 0)
    _ZT = RPT - CH * (RPT // CH)
    pltpu.sync_copy(ra0.at[pl.ds(0, _ZT)],
                    acc.at[pl.ds(s * RPT + CH * (RPT // CH), _ZT)])

    @pl.when(s == 0)
    def _():
        pltpu.sync_copy(ra0.at[pl.ds(0, TAIL)],
                        acc.at[pl.ds(NS * RPT, TAIL)])
    plsc.subcore_barrier()

    def _relu_add(ra, rb):
        def _cmp(i, cc):
            for j in range(8):
                sl = pl.ds(j * 16, 16)
                ra[i, sl] = jnp.maximum(ra[i, sl] + rb[i, sl], 0.0)
            return cc
        lax.fori_loop(0, CH, _cmp, 0)

    def _sup(sup, carry):
        pltpu.sync_copy(srcg.at[c, s, sup], idx_s2)
        pltpu.sync_copy(dstg.at[c, s, sup], idx_g2)
        pltpu.sync_copy(dstr.at[s, sup], idx_d2)
        pltpu.async_copy(ta.at[idx_g2.at[0]], ra0, semA0)
        pltpu.async_copy(tb.at[idx_s2.at[0]], rb0, semB0)

        def _pair(k, cc):
            g0 = 2 * k
            pltpu.async_copy(ta.at[idx_g2.at[g0 + 1]], ra1, semA1)
            pltpu.async_copy(tb.at[idx_s2.at[g0 + 1]], rb1, semB1)
            pltpu.make_async_copy(ta.at[idx_g2.at[g0]], ra0, semA0).wait()
            pltpu.make_async_copy(tb.at[idx_s2.at[g0]], rb0, semB0).wait()
            _relu_add(ra0, rb0)
            pltpu.sync_copy(ra0, acc.at[idx_d2.at[g0]], add=True)

            @pl.when(k < SPC // 2 - 1)
            def _():
                pltpu.async_copy(ta.at[idx_g2.at[g0 + 2]], ra0, semA0)
                pltpu.async_copy(tb.at[idx_s2.at[g0 + 2]], rb0, semB0)
            pltpu.make_async_copy(ta.at[idx_g2.at[g0 + 1]], ra1, semA1).wait()
            pltpu.make_async_copy(tb.at[idx_s2.at[g0 + 1]], rb1, semB1).wait()
            _relu_add(ra1, rb1)
            pltpu.sync_copy(ra1, acc.at[idx_d2.at[g0 + 1]], add=True)
            return cc
        lax.fori_loop(0, SPC // 2, _pair, 0)
        return carry

    lax.fori_loop(0, SUP, _sup, 0)
    plsc.subcore_barrier()

    def _piece(base, nr):
        pltpu.sync_copy(acc.at[pl.ds(base, nr)], rb0.at[pl.ds(0, nr)])
        pltpu.sync_copy(rb0.at[pl.ds(0, nr)], s_out.at[c, pl.ds(base, nr)])

    def _cok(k, carry):
        _piece(s * RPT + k * CP, CP)
        return carry
    lax.fori_loop(0, RPT // CP, _cok, 0)
    _piece(s * RPT + CP * (RPT // CP), RPT - CP * (RPT // CP))

    @pl.when(s == 0)
    def _():
        _piece(NS * RPT, TAIL)


def _make_edge_kernel():
    scratch = [
        pltpu.VMEM((SPC, CH), jnp.int32),
        pltpu.VMEM((SPC, CH), jnp.int32),
        pltpu.VMEM((SPC, CH), jnp.int32),
        pltpu.VMEM((CH, HALF), jnp.float32),
        pltpu.VMEM((CH, HALF), jnp.float32),
        pltpu.VMEM((CH, HALF), jnp.float32),
        pltpu.VMEM((CH, HALF), jnp.float32),
        pltpu.SemaphoreType.DMA,
        pltpu.SemaphoreType.DMA,
        pltpu.SemaphoreType.DMA,
        pltpu.SemaphoreType.DMA,
        pltpu.VMEM_SHARED((N, HALF), jnp.float32),
    ]
    mesh = plsc.VectorSubcoreMesh(core_axis_name="c", subcore_axis_name="s")
    return pl.kernel(
        _edge_body,
        out_type=jax.ShapeDtypeStruct((NC, N_PAD, HALF), jnp.float32),
        mesh=mesh,
        scratch_types=scratch,
        name="edge_scatter",
    )


def _count_body(dst, cnt_out, idx_d, ones, sem, cacc):
    c = lax.axis_index("c")
    s = lax.axis_index("s")
    zero16 = jnp.zeros((16,), jnp.float32)
    one16 = jnp.ones((16,), jnp.float32)

    def _zb(i, carry):
        for j in range(8):
            ones[i, pl.ds(j * 16, 16)] = zero16
        return carry
    lax.fori_loop(0, CCH, _zb, 0)

    def _za(k, carry):
        pltpu.sync_copy(ones, cacc.at[pl.ds(s * RPT + k * CCH, CCH)])
        return carry
    lax.fori_loop(0, RPT // CCH, _za, 0)
    _ZT = RPT - CCH * (RPT // CCH)
    pltpu.sync_copy(ones.at[pl.ds(0, _ZT)],
                    cacc.at[pl.ds(s * RPT + CCH * (RPT // CCH), _ZT)])

    @pl.when(s == 0)
    def _():
        pltpu.sync_copy(ones.at[pl.ds(0, TAIL)],
                        cacc.at[pl.ds(NS * RPT, TAIL)])

    def _ob(i, carry):
        for j in range(8):
            ones[i, pl.ds(j * 16, 16)] = one16
        return carry
    lax.fori_loop(0, CCH, _ob, 0)
    plsc.subcore_barrier()

    EHC = E // NC
    EPW2 = EHC // NS

    def _scat(base):
        pltpu.sync_copy(dst.at[pl.ds(base, CCH)], idx_d)
        pltpu.sync_copy(ones, cacc.at[idx_d], add=True)

    def _ck(k, carry):
        _scat(c * EHC + s * EPW2 + k * CCH)
        return carry
    lax.fori_loop(0, EPW2 // CCH, _ck, 0)
    _rem = EPW2 - CCH * (EPW2 // CCH)
    if _rem:
        def _zt(i, carry):
            for j in range(8):
                ones[i, pl.ds(j * 16, 16)] = zero16
            return carry
        lax.fori_loop(0, CCH - _rem, _zt, 0)
        _scat(c * EHC + (s + 1) * EPW2 - CH)
    plsc.subcore_barrier()

    def _piece(base, nr):
        pltpu.sync_copy(cacc.at[pl.ds(base, nr)], ones.at[pl.ds(0, nr)])
        pltpu.sync_copy(ones.at[pl.ds(0, nr)], cnt_out.at[c, pl.ds(base, nr)])

    def _cok(k, carry):
        _piece(s * RPT + k * CP, CP)
        return carry
    lax.fori_loop(0, RPT // CP, _cok, 0)
    _piece(s * RPT + CP * (RPT // CP), RPT - CP * (RPT // CP))

    @pl.when(s == 0)
    def _():
        _piece(NS * RPT, TAIL)


def _make_count_kernel():
    scratch = [
        pltpu.VMEM((CCH,), jnp.int32),
        pltpu.VMEM((CCH, HALF), jnp.float32),
        pltpu.SemaphoreType.DMA,
        pltpu.VMEM_SHARED((N, HALF), jnp.float32),
    ]
    mesh = plsc.VectorSubcoreMesh(core_axis_name="c", subcore_axis_name="s")
    return pl.kernel(
        _count_body,
        out_type=jax.ShapeDtypeStruct((NC, N_PAD, HALF), jnp.float32),
        mesh=mesh,
        scratch_types=scratch,
        name="edge_count",
    )


def _pre_body(h_ref, wa_ref, ba_ref, ta_ref, tb_ref):
    wa = wa_ref[...]
    wt = wa[:D] - wa[D:]
    wb = wa[D:]
    h = h_ref[...]
    a = jnp.dot(h, wt, preferred_element_type=jnp.float32) + ba_ref[...]
    b = jnp.dot(h, wb, preferred_element_type=jnp.float32)
    ta_ref[0] = a[:, :HALF]
    ta_ref[1] = a[:, HALF:]
    tb_ref[0] = b[:, :HALF]
    tb_ref[1] = b[:, HALF:]


_pre = pl.pallas_call(
    _pre_body,
    grid=(GRID,),
    in_specs=[
        pl.BlockSpec((TILE, D), lambda i: (i, 0)),
        pl.BlockSpec((2 * D, D), lambda i: (0, 0)),
        pl.BlockSpec((1, D), lambda i: (0, 0)),
    ],
    out_specs=[
        pl.BlockSpec((NC, TILE, HALF), lambda i: (0, i, 0)),
        pl.BlockSpec((NC, TILE, HALF), lambda i: (0, i, 0)),
    ],
    out_shape=[
        jax.ShapeDtypeStruct((NC, N_PAD, HALF), jnp.float32),
        jax.ShapeDtypeStruct((NC, N_PAD, HALF), jnp.float32),
    ],
)


def _post_body(s_ref, cnt_ref, wb_ref, bb_ref, o_ref):
    cv = cnt_ref[0][:, 0:1] + cnt_ref[1][:, 0:1]
    inv = 1.0 / jnp.maximum(cv, 1.0)
    wb = wb_ref[...]
    s0 = s_ref[0] * inv
    s1 = s_ref[1] * inv
    o = (jnp.dot(s0, wb[:HALF], preferred_element_type=jnp.float32)
         + jnp.dot(s1, wb[HALF:], preferred_element_type=jnp.float32))
    o_ref[...] = o + jnp.where(cv > 0.0, bb_ref[...], 0.0)


_post = pl.pallas_call(
    _post_body,
    grid=(GRID,),
    in_specs=[
        pl.BlockSpec((NC, TILE, HALF), lambda i: (0, i, 0)),
        pl.BlockSpec((NC, TILE, HALF), lambda i: (0, i, 0)),
        pl.BlockSpec((D, D), lambda i: (0, 0)),
        pl.BlockSpec((1, D), lambda i: (0, 0)),
    ],
    out_specs=pl.BlockSpec((TILE, D), lambda i: (i, 0)),
    out_shape=jax.ShapeDtypeStruct((N_PAD, D), jnp.float32),
)


def _heads_body(h_ref, wf1, bf1, wf2, bf2, ws1, bs1, ws2, bs2,
                wo1, bo1, wo2, bo2, wm1, bm1, wm2, bm2, o_ref):
    h = h_ref[...]

    def head(w1, b1, w2, b2):
        g = jnp.maximum(
            jnp.dot(h, w1[...], preferred_element_type=jnp.float32) + b1[...],
            0.0)
        return jnp.dot(g, w2[...], preferred_element_type=jnp.float32) + b2[...]

    f = head(wf1, bf1, wf2, bf2)
    sc = head(ws1, bs1, ws2, bs2)
    off = head(wo1, bo1, wo2, bo2)
    m = head(wm1, bm1, wm2, bm2)
    m = 1.0 / (1.0 + jnp.exp(-m))
    o_ref[...] = jnp.concatenate([f, sc, off, m], axis=-1)


def _w_spec(k):
    return pl.BlockSpec((D, k), lambda i: (0, 0))


def _b_spec(k):
    return pl.BlockSpec((1, k), lambda i: (0, 0))


_heads = pl.pallas_call(
    _heads_body,
    grid=(GRID,),
    in_specs=[
        pl.BlockSpec((TILE, D), lambda i: (i, 0)),
        _w_spec(D), _b_spec(D), _w_spec(32), _b_spec(32),
        _w_spec(D), _b_spec(D), _w_spec(6), _b_spec(6),
        _w_spec(D), _b_spec(D), _w_spec(30), _b_spec(30),
        _w_spec(D), _b_spec(D), _w_spec(11), _b_spec(11),
    ],
    out_specs=pl.BlockSpec((TILE, 79), lambda i: (i, 0)),
    out_shape=jax.ShapeDtypeStruct((N_PAD, 79), jnp.float32),
)


def kernel(x, edge_index, W1a, b1a, W1b, b1b, W2a, b2a, W2b, b2b,
           Wf1, bf1, Wf2, bf2, Ws1, bs1, Ws2, bs2,
           Wo1, bo1, Wo2, bo2, Wm1, bm1, Wm2, bm2):
    src = edge_index[0]
    dst = edge_index[1]
    r2 = lambda b: b.reshape(1, -1)
    xp = jnp.concatenate(
        [x, jnp.zeros((N_PAD - N, x.shape[1]), x.dtype)], axis=0)

    edge = _make_edge_kernel()
    count = _make_count_kernel()

    cntp = count(dst)
    src2 = jnp.concatenate([src, src + N_PAD]).reshape(NC, NS, SUP, SPC, CH)
    dst2 = jnp.concatenate([dst, dst + N_PAD]).reshape(NC, NS, SUP, SPC, CH)
    dst3 = dst.reshape(NS, SUP, SPC, CH)

    ta, tb = _pre(xp, W1a, r2(b1a))
    s_acc = edge(ta.reshape(NC * N_PAD, HALF),
                 tb.reshape(NC * N_PAD, HALF), src2, dst2, dst3)
    h = _post(s_acc, cntp, W1b, r2(b1b))

    ta, tb = _pre(h, W2a, r2(b2a))
    s_acc = edge(ta.reshape(NC * N_PAD, HALF),
                 tb.reshape(NC * N_PAD, HALF), src2, dst2, dst3)
    h = _post(s_acc, cntp, W2b, r2(b2b))

    return _heads(h, Wf1, r2(bf1), Wf2, r2(bf2), Ws1, r2(bs1), Ws2, r2(bs2),
                  Wo1, r2(bo1), Wo2, r2(bo2), Wm1, r2(bm1), Wm2, r2(bm2))[:N]


# speedup vs baseline: 1.1790x; 1.0263x over previous
import functools

import jax
import jax.numpy as jnp
from jax import lax
from jax.experimental import pallas as pl
from jax.experimental.pallas import tpu as pltpu
from jax.experimental.pallas import tpu_sc as plsc

N = 10000
E = 160000
D = 256
HALF = 128
NS = 16
NC = 2
EPW = E // NS
CH = 40
SUP = 5
SCH = 2000
SPC = SCH // CH
CCH = 200
NCHUNK = EPW // CH
RPT = 624
TAIL = N - NS * RPT
CP = 40
NW = NC * NS
ECW = E // NW
CROWS = 80
N_PAD = 10240
TILE = 1024
GRID = N_PAD // TILE


def _edge_body(ta, tb, srcg, dstg, dstr, s_out,
               idx_s2, idx_g2, idx_d2, ra0, rb0, ra1, rb1,
               semA0, semB0, semA1, semB1, acc):
    c = lax.axis_index("c")
    s = lax.axis_index("s")
    zero16 = jnp.zeros((16,), jnp.float32)

    def _zb(i, carry):
        for j in range(8):
            ra0[i, pl.ds(j * 16, 16)] = zero16
        return carry
    lax.fori_loop(0, CH, _zb, 0)

    def _za(k, carry):
        pltpu.sync_copy(ra0, acc.at[pl.ds(s * RPT + k * CH, CH)])
        return carry
    lax.fori_loop(0, RPT // CH, _za, 0)
    _ZT = RPT - CH * (RPT // CH)
    pltpu.sync_copy(ra0.at[pl.ds(0, _ZT)],
                    acc.at[pl.ds(s * RPT + CH * (RPT // CH), _ZT)])

    @pl.when(s == 0)
    def _():
        pltpu.sync_copy(ra0.at[pl.ds(0, TAIL)],
                        acc.at[pl.ds(NS * RPT, TAIL)])
    plsc.subcore_barrier()

    def _relu_add(ra, rb):
        def _cmp(i, cc):
            for j in range(8):
                sl = pl.ds(j * 16, 16)
                ra[i, sl] = jnp.maximum(ra[i, sl] + rb[i, sl], 0.0)
            return cc
        lax.fori_loop(0, CH, _cmp, 0)

    def _sup(sup, carry):
        pltpu.sync_copy(srcg.at[c, s, sup], idx_s2)
        pltpu.sync_copy(dstg.at[c, s, sup], idx_g2)
        pltpu.sync_copy(dstr.at[s, sup], idx_d2)
        pltpu.async_copy(ta.at[idx_g2.at[0]], ra0, semA0)
        pltpu.async_copy(tb.at[idx_s2.at[0]], rb0, semB0)

        def _pair(k, cc):
            g0 = 2 * k
            pltpu.async_copy(ta.at[idx_g2.at[g0 + 1]], ra1, semA1)
            pltpu.async_copy(tb.at[idx_s2.at[g0 + 1]], rb1, semB1)
            pltpu.make_async_copy(ta.at[idx_g2.at[g0]], ra0, semA0).wait()
            pltpu.make_async_copy(tb.at[idx_s2.at[g0]], rb0, semB0).wait()
            _relu_add(ra0, rb0)
            pltpu.sync_copy(ra0, acc.at[idx_d2.at[g0]], add=True)

            @pl.when(k < SPC // 2 - 1)
            def _():
                pltpu.async_copy(ta.at[idx_g2.at[g0 + 2]], ra0, semA0)
                pltpu.async_copy(tb.at[idx_s2.at[g0 + 2]], rb0, semB0)
            pltpu.make_async_copy(ta.at[idx_g2.at[g0 + 1]], ra1, semA1).wait()
            pltpu.make_async_copy(tb.at[idx_s2.at[g0 + 1]], rb1, semB1).wait()
            _relu_add(ra1, rb1)
            pltpu.sync_copy(ra1, acc.at[idx_d2.at[g0 + 1]], add=True)
            return cc
        lax.fori_loop(0, SPC // 2, _pair, 0)
        return carry

    lax.fori_loop(0, SUP, _sup, 0)
    plsc.subcore_barrier()

    def _piece(base, nr):
        pltpu.sync_copy(acc.at[pl.ds(base, nr)], rb0.at[pl.ds(0, nr)])
        pltpu.sync_copy(rb0.at[pl.ds(0, nr)], s_out.at[c, pl.ds(base, nr)])

    def _cok(k, carry):
        _piece(s * RPT + k * CP, CP)
        return carry
    lax.fori_loop(0, RPT // CP, _cok, 0)
    _piece(s * RPT + CP * (RPT // CP), RPT - CP * (RPT // CP))

    @pl.when(s == 0)
    def _():
        _piece(NS * RPT, TAIL)


def _make_edge_kernel():
    scratch = [
        pltpu.VMEM((SPC, CH), jnp.int32),
        pltpu.VMEM((SPC, CH), jnp.int32),
        pltpu.VMEM((SPC, CH), jnp.int32),
        pltpu.VMEM((CH, HALF), jnp.float32),
        pltpu.VMEM((CH, HALF), jnp.float32),
        pltpu.VMEM((CH, HALF), jnp.float32),
        pltpu.VMEM((CH, HALF), jnp.float32),
        pltpu.SemaphoreType.DMA,
        pltpu.SemaphoreType.DMA,
        pltpu.SemaphoreType.DMA,
        pltpu.SemaphoreType.DMA,
        pltpu.VMEM_SHARED((N, HALF), jnp.float32),
    ]
    mesh = plsc.VectorSubcoreMesh(core_axis_name="c", subcore_axis_name="s")
    return pl.kernel(
        _edge_body,
        out_type=jax.ShapeDtypeStruct((NC, N_PAD, HALF), jnp.float32),
        mesh=mesh,
        scratch_types=scratch,
        name="edge_scatter",
    )


def _count_body(dst, cnt_out, idx_d, ones, sem, cacc):
    c = lax.axis_index("c")
    s = lax.axis_index("s")
    zero16 = jnp.zeros((16,), jnp.float32)
    one16 = jnp.ones((16,), jnp.float32)

    def _zb(i, carry):
        for j in range(8):
            ones[i, pl.ds(j * 16, 16)] = zero16
        return carry
    lax.fori_loop(0, CCH, _zb, 0)

    def _za(k, carry):
        pltpu.sync_copy(ones, cacc.at[pl.ds(s * RPT + k * CCH, CCH)])
        return carry
    lax.fori_loop(0, RPT // CCH, _za, 0)
    _ZT = RPT - CCH * (RPT // CCH)
    pltpu.sync_copy(ones.at[pl.ds(0, _ZT)],
                    cacc.at[pl.ds(s * RPT + CCH * (RPT // CCH), _ZT)])

    @pl.when(s == 0)
    def _():
        pltpu.sync_copy(ones.at[pl.ds(0, TAIL)],
                        cacc.at[pl.ds(NS * RPT, TAIL)])

    def _ob(i, carry):
        for j in range(8):
            ones[i, pl.ds(j * 16, 16)] = one16
        return carry
    lax.fori_loop(0, CCH, _ob, 0)
    plsc.subcore_barrier()

    EHC = E // NC
    EPW2 = EHC // NS

    def _scat(base):
        pltpu.sync_copy(dst.at[pl.ds(base, CCH)], idx_d)
        pltpu.sync_copy(ones, cacc.at[idx_d], add=True)

    def _ck(k, carry):
        _scat(c * EHC + s * EPW2 + k * CCH)
        return carry
    lax.fori_loop(0, EPW2 // CCH, _ck, 0)
    _rem = EPW2 - CCH * (EPW2 // CCH)
    if _rem:
        def _zt(i, carry):
            for j in range(8):
                ones[i, pl.ds(j * 16, 16)] = zero16
            return carry
        lax.fori_loop(0, CCH - _rem, _zt, 0)
        _scat(c * EHC + (s + 1) * EPW2 - CH)
    plsc.subcore_barrier()

    def _piece(base, nr):
        pltpu.sync_copy(cacc.at[pl.ds(base, nr)], ones.at[pl.ds(0, nr)])
        pltpu.sync_copy(ones.at[pl.ds(0, nr)], cnt_out.at[c, pl.ds(base, nr)])

    def _cok(k, carry):
        _piece(s * RPT + k * CP, CP)
        return carry
    lax.fori_loop(0, RPT // CP, _cok, 0)
    _piece(s * RPT + CP * (RPT // CP), RPT - CP * (RPT // CP))

    @pl.when(s == 0)
    def _():
        _piece(NS * RPT, TAIL)


def _make_count_kernel():
    scratch = [
        pltpu.VMEM((CCH,), jnp.int32),
        pltpu.VMEM((CCH, HALF), jnp.float32),
        pltpu.SemaphoreType.DMA,
        pltpu.VMEM_SHARED((N, HALF), jnp.float32),
    ]
    mesh = plsc.VectorSubcoreMesh(core_axis_name="c", subcore_axis_name="s")
    return pl.kernel(
        _count_body,
        out_type=jax.ShapeDtypeStruct((NC, N_PAD, HALF), jnp.float32),
        mesh=mesh,
        scratch_types=scratch,
        name="edge_count",
    )


def _pre_body(h_ref, wa_ref, ba_ref, ta_ref, tb_ref):
    wa = wa_ref[...]
    wt = wa[:D] - wa[D:]
    wb = wa[D:]
    h = h_ref[...]
    a = jnp.dot(h, wt, preferred_element_type=jnp.float32) + ba_ref[...]
    b = jnp.dot(h, wb, preferred_element_type=jnp.float32)
    ta_ref[0] = a[:, :HALF]
    ta_ref[1] = a[:, HALF:]
    tb_ref[0] = b[:, :HALF]
    tb_ref[1] = b[:, HALF:]


_pre = pl.pallas_call(
    _pre_body,
    grid=(GRID,),
    in_specs=[
        pl.BlockSpec((TILE, D), lambda i: (i, 0)),
        pl.BlockSpec((2 * D, D), lambda i: (0, 0)),
        pl.BlockSpec((1, D), lambda i: (0, 0)),
    ],
    out_specs=[
        pl.BlockSpec((NC, TILE, HALF), lambda i: (0, i, 0)),
        pl.BlockSpec((NC, TILE, HALF), lambda i: (0, i, 0)),
    ],
    out_shape=[
        jax.ShapeDtypeStruct((NC, N_PAD, HALF), jnp.float32),
        jax.ShapeDtypeStruct((NC, N_PAD, HALF), jnp.float32),
    ],
)


def _post_body(s_ref, cnt_ref, wb_ref, bb_ref, o_ref):
    cv = cnt_ref[0][:, 0:1] + cnt_ref[1][:, 0:1]
    inv = 1.0 / jnp.maximum(cv, 1.0)
    wb = wb_ref[...]
    s0 = s_ref[0] * inv
    s1 = s_ref[1] * inv
    o = (jnp.dot(s0, wb[:HALF], preferred_element_type=jnp.float32)
         + jnp.dot(s1, wb[HALF:], preferred_element_type=jnp.float32))
    o_ref[...] = o + jnp.where(cv > 0.0, bb_ref[...], 0.0)


_post = pl.pallas_call(
    _post_body,
    grid=(GRID,),
    in_specs=[
        pl.BlockSpec((NC, TILE, HALF), lambda i: (0, i, 0)),
        pl.BlockSpec((NC, TILE, HALF), lambda i: (0, i, 0)),
        pl.BlockSpec((D, D), lambda i: (0, 0)),
        pl.BlockSpec((1, D), lambda i: (0, 0)),
    ],
    out_specs=pl.BlockSpec((TILE, D), lambda i: (i, 0)),
    out_shape=jax.ShapeDtypeStruct((N_PAD, D), jnp.float32),
)


def _heads_body(h_ref, wf1, bf1, wf2, bf2, ws1, bs1, ws2, bs2,
                wo1, bo1, wo2, bo2, wm1, bm1, wm2, bm2, o_ref):
    h = h_ref[...]

    def head(w1, b1, w2, b2):
        g = jnp.maximum(
            jnp.dot(h, w1[...], preferred_element_type=jnp.float32) + b1[...],
            0.0)
        return jnp.dot(g, w2[...], preferred_element_type=jnp.float32) + b2[...]

    f = head(wf1, bf1, wf2, bf2)
    sc = head(ws1, bs1, ws2, bs2)
    off = head(wo1, bo1, wo2, bo2)
    m = head(wm1, bm1, wm2, bm2)
    m = 1.0 / (1.0 + jnp.exp(-m))
    o_ref[...] = jnp.concatenate([f, sc, off, m], axis=-1)


def _w_spec(k):
    return pl.BlockSpec((D, k), lambda i: (0, 0))


def _b_spec(k):
    return pl.BlockSpec((1, k), lambda i: (0, 0))


_heads = pl.pallas_call(
    _heads_body,
    grid=(GRID,),
    in_specs=[
        pl.BlockSpec((TILE, D), lambda i: (i, 0)),
        _w_spec(D), _b_spec(D), _w_spec(32), _b_spec(32),
        _w_spec(D), _b_spec(D), _w_spec(6), _b_spec(6),
        _w_spec(D), _b_spec(D), _w_spec(30), _b_spec(30),
        _w_spec(D), _b_spec(D), _w_spec(11), _b_spec(11),
    ],
    out_specs=pl.BlockSpec((TILE, 79), lambda i: (i, 0)),
    out_shape=jax.ShapeDtypeStruct((N_PAD, 79), jnp.float32),
)


def _mid_body(s_ref, cnt_ref, wb_ref, bb_ref, wa_ref, ba_ref,
              ta_ref, tb_ref):
    cv = cnt_ref[0][:, 0:1] + cnt_ref[1][:, 0:1]
    inv = 1.0 / jnp.maximum(cv, 1.0)
    wb = wb_ref[...]
    h = (jnp.dot(s_ref[0] * inv, wb[:HALF], preferred_element_type=jnp.float32)
         + jnp.dot(s_ref[1] * inv, wb[HALF:],
                   preferred_element_type=jnp.float32))
    h = h + jnp.where(cv > 0.0, bb_ref[...], 0.0)
    wa = wa_ref[...]
    wt = wa[:D] - wa[D:]
    wb2 = wa[D:]
    a = jnp.dot(h, wt, preferred_element_type=jnp.float32) + ba_ref[...]
    b = jnp.dot(h, wb2, preferred_element_type=jnp.float32)
    ta_ref[0] = a[:, :HALF]
    ta_ref[1] = a[:, HALF:]
    tb_ref[0] = b[:, :HALF]
    tb_ref[1] = b[:, HALF:]


_mid = pl.pallas_call(
    _mid_body,
    grid=(GRID,),
    in_specs=[
        pl.BlockSpec((NC, TILE, HALF), lambda i: (0, i, 0)),
        pl.BlockSpec((NC, TILE, HALF), lambda i: (0, i, 0)),
        pl.BlockSpec((D, D), lambda i: (0, 0)),
        pl.BlockSpec((1, D), lambda i: (0, 0)),
        pl.BlockSpec((2 * D, D), lambda i: (0, 0)),
        pl.BlockSpec((1, D), lambda i: (0, 0)),
    ],
    out_specs=[
        pl.BlockSpec((NC, TILE, HALF), lambda i: (0, i, 0)),
        pl.BlockSpec((NC, TILE, HALF), lambda i: (0, i, 0)),
    ],
    out_shape=[
        jax.ShapeDtypeStruct((NC, N_PAD, HALF), jnp.float32),
        jax.ShapeDtypeStruct((NC, N_PAD, HALF), jnp.float32),
    ],
)


def _fin_body(s_ref, cnt_ref, wb_ref, bb_ref,
              wf1, bf1, wf2, bf2, ws1, bs1, ws2, bs2,
              wo1, bo1, wo2, bo2, wm1, bm1, wm2, bm2, o_ref):
    cv = cnt_ref[0][:, 0:1] + cnt_ref[1][:, 0:1]
    inv = 1.0 / jnp.maximum(cv, 1.0)
    wb = wb_ref[...]
    h = (jnp.dot(s_ref[0] * inv, wb[:HALF], preferred_element_type=jnp.float32)
         + jnp.dot(s_ref[1] * inv, wb[HALF:],
                   preferred_element_type=jnp.float32))
    h = h + jnp.where(cv > 0.0, bb_ref[...], 0.0)

    def head(w1, b1, w2, b2):
        g = jnp.maximum(
            jnp.dot(h, w1[...], preferred_element_type=jnp.float32) + b1[...],
            0.0)
        return jnp.dot(g, w2[...], preferred_element_type=jnp.float32) + b2[...]

    f = head(wf1, bf1, wf2, bf2)
    sc = head(ws1, bs1, ws2, bs2)
    off = head(wo1, bo1, wo2, bo2)
    m = head(wm1, bm1, wm2, bm2)
    m = 1.0 / (1.0 + jnp.exp(-m))
    o_ref[...] = jnp.concatenate([f, sc, off, m], axis=-1)


_fin = pl.pallas_call(
    _fin_body,
    grid=(GRID,),
    in_specs=[
        pl.BlockSpec((NC, TILE, HALF), lambda i: (0, i, 0)),
        pl.BlockSpec((NC, TILE, HALF), lambda i: (0, i, 0)),
        pl.BlockSpec((D, D), lambda i: (0, 0)),
        pl.BlockSpec((1, D), lambda i: (0, 0)),
        _w_spec(D), _b_spec(D), _w_spec(32), _b_spec(32),
        _w_spec(D), _b_spec(D), _w_spec(6), _b_spec(6),
        _w_spec(D), _b_spec(D), _w_spec(30), _b_spec(30),
        _w_spec(D), _b_spec(D), _w_spec(11), _b_spec(11),
    ],
    out_specs=pl.BlockSpec((TILE, 79), lambda i: (i, 0)),
    out_shape=jax.ShapeDtypeStruct((N_PAD, 79), jnp.float32),
)


def kernel(x, edge_index, W1a, b1a, W1b, b1b, W2a, b2a, W2b, b2b,
           Wf1, bf1, Wf2, bf2, Ws1, bs1, Ws2, bs2,
           Wo1, bo1, Wo2, bo2, Wm1, bm1, Wm2, bm2):
    src = edge_index[0]
    dst = edge_index[1]
    r2 = lambda b: b.reshape(1, -1)
    xp = jnp.concatenate(
        [x, jnp.zeros((N_PAD - N, x.shape[1]), x.dtype)], axis=0)

    edge = _make_edge_kernel()
    count = _make_count_kernel()

    cntp = count(dst)
    src2 = jnp.concatenate([src, src + N_PAD]).reshape(NC, NS, SUP, SPC, CH)
    dst2 = jnp.concatenate([dst, dst + N_PAD]).reshape(NC, NS, SUP, SPC, CH)
    dst3 = dst.reshape(NS, SUP, SPC, CH)

    ta, tb = _pre(xp, W1a, r2(b1a))
    s_acc = edge(ta.reshape(NC * N_PAD, HALF),
                 tb.reshape(NC * N_PAD, HALF), src2, dst2, dst3)
    ta, tb = _mid(s_acc, cntp, W1b, r2(b1b), W2a, r2(b2a))
    s_acc = edge(ta.reshape(NC * N_PAD, HALF),
                 tb.reshape(NC * N_PAD, HALF), src2, dst2, dst3)
    return _fin(s_acc, cntp, W2b, r2(b2b),
                Wf1, r2(bf1), Wf2, r2(bf2), Ws1, r2(bs1), Ws2, r2(bs2),
                Wo1, r2(bo1), Wo2, r2(bo2), Wm1, r2(bm1), Wm2, r2(bm2))[:N]
